# Initial kernel scaffold; baseline (speedup 1.0000x reference)
import jax
import jax.numpy as jnp
from jax.experimental import pallas as pl


def kernel(inputs, shared_weights):
    raise NotImplementedError("write your pallas kernel here")


# speedup vs baseline: 3.2771x; 3.2771x over previous
import functools

import jax
import jax.numpy as jnp
from jax import lax
from jax.experimental import pallas as pl
from jax.experimental.pallas import tpu as pltpu
from jax.experimental.pallas import tpu_sc as plsc

_VOCAB = 100000
_EMBED = 64
_SCALE = 8.0

_B = 4096
_T = 200
_N = _B * _T

_NW = 32
_CH = 128
_G = _N // (_NW * _CH)

_PREP_ROWS = 2000


def _prep_body(w_ref, o_ref):
    base = pl.program_id(0) * _PREP_ROWS
    rid = lax.broadcasted_iota(jnp.int32, (_PREP_ROWS, _EMBED), 0) + base
    o_ref[...] = jnp.where(rid == 0, 0.0, w_ref[...] * _SCALE)


def _prep_table(w):
    return pl.pallas_call(
        _prep_body,
        grid=(_VOCAB // _PREP_ROWS,),
        in_specs=[pl.BlockSpec((_PREP_ROWS, _EMBED), lambda i: (i, 0))],
        out_specs=pl.BlockSpec((_PREP_ROWS, _EMBED), lambda i: (i, 0)),
        out_shape=jax.ShapeDtypeStruct((_VOCAB, _EMBED), jnp.float32),
    )(w)


_mesh = plsc.VectorSubcoreMesh(core_axis_name="c", subcore_axis_name="s")


@functools.partial(
    pl.kernel,
    mesh=_mesh,
    out_type=jax.ShapeDtypeStruct((_NW, _G, _CH, _EMBED), jnp.float32),
    scratch_types=[
        pltpu.VMEM((_G, _CH), jnp.int32),
        pltpu.VMEM((_CH, _EMBED), jnp.float32),
        pltpu.SemaphoreType.DMA,
    ],
    compiler_params=pltpu.CompilerParams(use_tc_tiling_on_sc=False),
)
def _sc_gather(idx_hbm, table_hbm, out_hbm, idx_v, rows_v, sem):
    wid = lax.axis_index("s") * 2 + lax.axis_index("c")
    pltpu.sync_copy(idx_hbm.at[wid], idx_v)

    def step(g, carry):
        pltpu.async_copy(table_hbm.at[idx_v.at[g]], rows_v, sem).wait()
        pltpu.sync_copy(rows_v, out_hbm.at[wid, g])
        return carry

    lax.fori_loop(0, _G, step, 0)


def kernel(inputs, shared_weights):
    table = _prep_table(shared_weights)
    idx = inputs.astype(jnp.int32).reshape(_NW, _G, _CH)
    out = _sc_gather(idx, table)
    return out.reshape(_B, _T, _EMBED)


# speedup vs baseline: 3.8725x; 1.1817x over previous
import functools

import jax
import jax.numpy as jnp
from jax import lax
from jax.experimental import pallas as pl
from jax.experimental.pallas import tpu as pltpu
from jax.experimental.pallas import tpu_sc as plsc

_VOCAB = 100000
_EMBED = 64
_SCALE = 8.0

_B = 4096
_T = 200
_N = _B * _T

_NW = 32
_CH = 128
_G = _N // (_NW * _CH)

_PREP_ROWS = 2000


def _prep_body(w_ref, o_ref):
    base = pl.program_id(0) * _PREP_ROWS
    rid = lax.broadcasted_iota(jnp.int32, (_PREP_ROWS, _EMBED), 0) + base
    o_ref[...] = jnp.where(rid == 0, 0.0, w_ref[...] * _SCALE)


def _prep_table(w):
    return pl.pallas_call(
        _prep_body,
        grid=(_VOCAB // _PREP_ROWS,),
        in_specs=[pl.BlockSpec((_PREP_ROWS, _EMBED), lambda i: (i, 0))],
        out_specs=pl.BlockSpec((_PREP_ROWS, _EMBED), lambda i: (i, 0)),
        out_shape=jax.ShapeDtypeStruct((_VOCAB, _EMBED), jnp.float32),
    )(w)


_mesh = plsc.VectorSubcoreMesh(core_axis_name="c", subcore_axis_name="s")

_NB = 4


@functools.partial(
    pl.kernel,
    mesh=_mesh,
    out_type=jax.ShapeDtypeStruct((_NW, _G, _CH, _EMBED), jnp.float32),
    scratch_types=[
        pltpu.VMEM((_G, _CH), jnp.int32),
        pltpu.VMEM((_NB, _CH, _EMBED), jnp.float32),
    ]
    + [pltpu.SemaphoreType.DMA] * (2 * _NB),
    compiler_params=pltpu.CompilerParams(use_tc_tiling_on_sc=False),
)
def _sc_gather(idx_hbm, table_hbm, out_hbm, idx_v, rows_v, *sems):
    gsem, osem = sems[:_NB], sems[_NB:]
    wid = lax.axis_index("s") * 2 + lax.axis_index("c")
    pltpu.sync_copy(idx_hbm.at[wid], idx_v)

    def _wait_gather(b):
        pltpu.make_async_copy(
            table_hbm.at[idx_v.at[b]], rows_v.at[b], gsem[b]
        ).wait()

    def _wait_out(b):
        pltpu.make_async_copy(rows_v.at[b], out_hbm.at[wid, 0], osem[b]).wait()

    for b in range(_NB):
        pltpu.async_copy(table_hbm.at[idx_v.at[b]], rows_v.at[b], gsem[b])

    def _step(s, b, issue_next):
        g = s * _NB + b
        bp = (b - 1) % _NB
        if issue_next:
            _wait_out(bp)
            pltpu.async_copy(
                table_hbm.at[idx_v.at[g + _NB - 1]], rows_v.at[bp], gsem[bp]
            )
        _wait_gather(b)
        pltpu.async_copy(rows_v.at[b], out_hbm.at[wid, g], osem[b])

    _step(0, 0, False)
    for b in range(1, _NB):
        _step(0, b, True)

    def outer(s, carry):
        for b in range(_NB):
            _step(s, b, True)
        return carry

    lax.fori_loop(1, _G // _NB - 1, outer, 0)

    s_last = _G // _NB - 1
    _step(s_last, 0, True)
    for b in range(1, _NB):
        _step(s_last, b, False)

    for b in range(_NB):
        _wait_out(b)


def kernel(inputs, shared_weights):
    table = _prep_table(shared_weights)
    idx = inputs.astype(jnp.int32).reshape(_NW, _G, _CH)
    out = _sc_gather(idx, table)
    return out.reshape(_B, _T, _EMBED)
